# baseline (speedup 1.0000x reference)
import functools

import jax
import jax.numpy as jnp
from jax import lax
from jax.experimental import pallas as pl
from jax.experimental.pallas import tpu as pltpu
from jax.experimental.pallas import tpu_sc as plsc

IMG = 512.0
IOU_THR = 0.5
SCORE_THR = 0.05
MAX_OUT = 64
NUM_CLASSES = 80
N = 20000
NEG = -1e9

L = 16
CH0 = 128
NP = 20480
NL1 = NP // CH0
NL2 = L
OUTW = 5 * MAX_OUT

NUM_SC_CORES = 2
NUM_SUBCORES = 16
NW = NUM_SC_CORES * NUM_SUBCORES
CLASSES_PER_W = (NUM_CLASSES + NW - 1) // NW


def _prep_tc_kernel(s_ref, anc_ref, reg_ref, lvl1_ref, lvl2_ref, boxes_ref):
    s = s_ref[...]
    sp = jnp.concatenate(
        [s, jnp.full((NUM_CLASSES, NP - N), NEG, jnp.float32)], axis=1)
    c1 = jnp.max(sp.reshape(NUM_CLASSES, NL1, CH0), axis=2)
    l1 = jnp.where(c1 > SCORE_THR, c1, NEG)
    lvl1_ref[...] = l1
    c2 = jnp.max(l1.reshape(NUM_CLASSES, 10, L), axis=2)
    lvl2_ref[...] = jnp.concatenate(
        [c2, jnp.full((NUM_CLASSES, NL2 - 10), NEG, jnp.float32)], axis=1)
    a0 = anc_ref[0, :]
    a1 = anc_ref[1, :]
    a2 = anc_ref[2, :]
    a3 = anc_ref[3, :]
    r0 = reg_ref[0, :]
    r1 = reg_ref[1, :]
    r2 = reg_ref[2, :]
    r3 = reg_ref[3, :]
    w = a2 - a0
    h = a3 - a1
    cx = a0 + 0.5 * w
    cy = a1 + 0.5 * h
    dx = r0 * 0.1
    dy = r1 * 0.1
    dw = r2 * 0.2
    dh = r3 * 0.2
    pcx = cx + dx * w
    pcy = cy + dy * h
    pw = jnp.exp(dw) * w
    ph = jnp.exp(dh) * h
    boxes_ref[0, :] = jnp.maximum(pcx - 0.5 * pw, 0.0)
    boxes_ref[1, :] = jnp.maximum(pcy - 0.5 * ph, 0.0)
    boxes_ref[2, :] = jnp.minimum(pcx + 0.5 * pw, IMG)
    boxes_ref[3, :] = jnp.minimum(pcy + 0.5 * ph, IMG)


def _prep(scores_t, anchors_t, regression_t):
    return pl.pallas_call(
        _prep_tc_kernel,
        out_shape=[
            jax.ShapeDtypeStruct((NUM_CLASSES, NL1), jnp.float32),
            jax.ShapeDtypeStruct((NUM_CLASSES, NL2), jnp.float32),
            jax.ShapeDtypeStruct((4, N), jnp.float32),
        ],
    )(scores_t, anchors_t, regression_t)


def _smax(v, perms):
    for p in perms:
        v = jnp.maximum(v, jnp.take(v, p))
    return v


def _sc_nms_kernel(scores_hbm, lvl1_hbm, lvl2_hbm, boxes_hbm, out_hbm,
                   sc_a, sc_b, l1_a, l1_b, l2_a, l2_b,
                   bx1_v, by1_v, bx2_v, by2_v,
                   sxa1, sya1, sxa2, sya2, osa,
                   sxb1, syb1, sxb2, syb2, osb,
                   stg0, stg1, stg2,
                   semb, sem0, sem1, semo):
    wid = lax.axis_index("s") * NUM_SC_CORES + lax.axis_index("c")
    iot = lax.iota(jnp.int32, L)
    perms = [iot ^ (1 << k) for k in range(4)]
    zeros = jnp.zeros((L,), jnp.float32)
    negv = jnp.full((L,), NEG, jnp.float32)
    thrv = jnp.full((L,), SCORE_THR, jnp.float32)
    iouv = jnp.full((L,), IOU_THR, jnp.float32)

    bufs = [(sc_a, l1_a, l2_a, sem0), (sc_b, l1_b, l2_b, sem1)]
    sels = [(sxa1, sya1, sxa2, sya2, osa), (sxb1, syb1, sxb2, syb2, osb)]
    stgs = [stg0, stg1, stg2]

    def fire(c, t):
        sc_v, l1_v, l2_v, sem = bufs[t % 2]
        return [pltpu.async_copy(scores_hbm.at[c], sc_v, sem),
                pltpu.async_copy(lvl1_hbm.at[c], l1_v, sem),
                pltpu.async_copy(lvl2_hbm.at[c], l2_v, sem)]

    def zero_sel(sel):
        for ref in sel:
            for k in range(MAX_OUT // L):
                ref[pl.ds(k * L, L)] = zeros

    def half(count, mv, t, si):
        sc_v, l1_v, l2_v, _ = bufs[t % 2]
        sx1_v, sy1_v, sx2_v, sy2_v, os_v = sels[si]
        active = jnp.logical_and(count < MAX_OUT, jnp.any(mv > thrv))
        activv = jnp.full((L,), active)
        v2 = l2_v[...]
        j2 = jnp.minimum(plsc.all_reduce_ffs(v2 == mv)[0], NL2 - 7)
        v1 = l1_v[pl.ds(j2 * L, L)]
        j1loc = plsc.all_reduce_ffs(v1 == mv)[0]
        j1 = jnp.minimum(j2 * L + j1loc, N // CH0)
        base0 = j1 * CH0
        r2v = _smax(jnp.where(iot == j2, negv, v2), perms)
        r1v = _smax(jnp.where(iot == j1loc, negv, v1), perms)
        offs = [jnp.minimum(base0 + k * L, N - L) for k in range(CH0 // L)]
        subs = []
        cand = jnp.full((L,), 4096, jnp.int32)
        for k in range(CH0 // L):
            sk = sc_v[pl.ds(offs[k], L)]
            subs.append(sk)
            f = plsc.all_reduce_ffs(sk == mv)
            cand = jnp.minimum(
                cand, jnp.where(f == L, 4096, k * L + f))
        idx_loc = cand[0]
        lane = lax.rem(idx_loc, L)
        k0 = lax.div(idx_loc, L)
        off0 = jnp.minimum(base0 + k0 * L, N - L)
        idx = jnp.minimum(base0 + idx_loc, N - 1)
        sub = sc_v[pl.ds(off0, L)]
        subn = jnp.where(jnp.logical_and(iot == lane, activv), negv, sub)
        sc_v[pl.ds(off0, L)] = subn
        part = jnp.where(jnp.full((L,), offs[0] == off0), subn, subs[0])
        for k in range(1, CH0 // L):
            sk = jnp.where(jnp.full((L,), offs[k] == off0), subn, subs[k])
            part = jnp.maximum(part, sk)
        m1v_raw = _smax(part, perms)
        m1v = jnp.where(m1v_raw > thrv, m1v_raw, negv)
        v1n = jnp.where(jnp.logical_and(iot == j1loc, activv), m1v, v1)
        l1_v[pl.ds(j2 * L, L)] = v1n
        m2v = jnp.maximum(m1v, r1v)
        v2n = jnp.where(jnp.logical_and(iot == j2, activv), m2v, v2)
        l2_v[...] = v2n
        m_next = jnp.where(activv, jnp.maximum(m2v, r2v), mv)
        idxv = jnp.full((L,), idx, jnp.int32)
        cx1 = plsc.load_gather(bx1_v, [idxv])
        cy1 = plsc.load_gather(by1_v, [idxv])
        cx2 = plsc.load_gather(bx2_v, [idxv])
        cy2 = plsc.load_gather(by2_v, [idxv])
        ca = (cx2 - cx1) * (cy2 - cy1)
        acc = zeros
        for k in range(MAX_OUT // L):
            sx1 = sx1_v[pl.ds(k * L, L)]
            sy1 = sy1_v[pl.ds(k * L, L)]
            sx2 = sx2_v[pl.ds(k * L, L)]
            sy2 = sy2_v[pl.ds(k * L, L)]
            xx1 = jnp.maximum(sx1, cx1)
            yy1 = jnp.maximum(sy1, cy1)
            xx2 = jnp.minimum(sx2, cx2)
            yy2 = jnp.minimum(sy2, cy2)
            inter = jnp.maximum(xx2 - xx1, 0.0) * jnp.maximum(yy2 - yy1, 0.0)
            sa = (sx2 - sx1) * (sy2 - sy1)
            iou = inter / (ca + sa - inter + 1e-8)
            acc = jnp.maximum(acc, iou)
        ok = jnp.logical_and(jnp.logical_not(jnp.any(acc > iouv)), active)
        coff = jnp.minimum(lax.div(count, L) * L, MAX_OUT - L)
        lmask = jnp.logical_and(iot == (count - coff),
                                jnp.full((L,), ok))
        for ref, val in ((sx1_v, cx1), (sy1_v, cy1),
                         (sx2_v, cx2), (sy2_v, cy2), (os_v, mv)):
            w = ref[pl.ds(coff, L)]
            ref[pl.ds(coff, L)] = jnp.where(lmask, val, w)
        return count + ok.astype(jnp.int32), m_next

    def stage_out(c, t, si):
        stg = stgs[t]
        sx1_v, sy1_v, sx2_v, sy2_v, os_v = sels[si]
        for i, ref in enumerate((os_v, sx1_v, sy1_v, sx2_v, sy2_v)):
            for k in range(MAX_OUT // L):
                stg[pl.ds(i * MAX_OUT + k * L, L)] = ref[pl.ds(k * L, L)]
        return pltpu.async_copy(stg, out_hbm.at[c], semo)

    def walk_pair(cA, cB):
        zero_sel(sels[0])
        zero_sel(sels[1])
        m0a = _smax(l2_a[...], perms)
        m0b = _smax(l2_b[...], perms)

        def cond(carry):
            ca_, mva, cb_, mvb = carry
            a = jnp.logical_and(ca_ < MAX_OUT, jnp.any(mva > thrv))
            b = jnp.logical_and(cb_ < MAX_OUT, jnp.any(mvb > thrv))
            return jnp.logical_or(a, b)

        def body(carry):
            ca_, mva, cb_, mvb = carry
            ca2, mva2 = half(ca_, mva, 0, 0)
            cb2, mvb2 = half(cb_, mvb, 1, 1)
            return ca2, mva2, cb2, mvb2

        lax.while_loop(cond, body, (jnp.int32(0), m0a, jnp.int32(0), m0b))
        return stage_out(cA, 0, 0), stage_out(cB, 1, 1)

    def walk_single(c):
        zero_sel(sels[0])
        m0 = _smax(l2_a[...], perms)

        def cond(carry):
            count, mv = carry
            return jnp.logical_and(count < MAX_OUT, jnp.any(mv > thrv))

        def body(carry):
            return half(carry[0], carry[1], 0, 0)

        lax.while_loop(cond, body, (jnp.int32(0), m0))
        return stage_out(c, 2, 0)

    hb = [pltpu.async_copy(boxes_hbm.at[0], bx1_v, semb),
          pltpu.async_copy(boxes_hbm.at[1], by1_v, semb),
          pltpu.async_copy(boxes_hbm.at[2], bx2_v, semb),
          pltpu.async_copy(boxes_hbm.at[3], by2_v, semb)]
    h0 = fire(wid, 0)
    h1 = fire(wid + NW, 1)
    for h in hb + h0 + h1:
        h.wait()

    ho0, ho1 = walk_pair(wid, wid + NW)

    has3 = wid + 2 * NW < NUM_CLASSES
    c2 = jnp.minimum(wid + 2 * NW, NUM_CLASSES - 1)
    h2 = fire(c2, 0)
    for h in h2:
        h.wait()

    @pl.when(has3)
    def _():
        walk_single(wid + 2 * NW).wait()
    ho0.wait()
    ho1.wait()


@functools.partial(
    pl.kernel,
    out_type=jax.ShapeDtypeStruct((NUM_CLASSES, OUTW), jnp.float32),
    mesh=plsc.VectorSubcoreMesh(core_axis_name="c", subcore_axis_name="s",
                                num_cores=NUM_SC_CORES,
                                num_subcores=NUM_SUBCORES),
    compiler_params=pltpu.CompilerParams(needs_layout_passes=False),
    scratch_types=[
        pltpu.VMEM((N,), jnp.float32),
        pltpu.VMEM((N,), jnp.float32),
        pltpu.VMEM((NL1,), jnp.float32),
        pltpu.VMEM((NL1,), jnp.float32),
        pltpu.VMEM((NL2,), jnp.float32),
        pltpu.VMEM((NL2,), jnp.float32),
        pltpu.VMEM((N,), jnp.float32),
        pltpu.VMEM((N,), jnp.float32),
        pltpu.VMEM((N,), jnp.float32),
        pltpu.VMEM((N,), jnp.float32),
        pltpu.VMEM((MAX_OUT,), jnp.float32),
        pltpu.VMEM((MAX_OUT,), jnp.float32),
        pltpu.VMEM((MAX_OUT,), jnp.float32),
        pltpu.VMEM((MAX_OUT,), jnp.float32),
        pltpu.VMEM((MAX_OUT,), jnp.float32),
        pltpu.VMEM((MAX_OUT,), jnp.float32),
        pltpu.VMEM((MAX_OUT,), jnp.float32),
        pltpu.VMEM((MAX_OUT,), jnp.float32),
        pltpu.VMEM((MAX_OUT,), jnp.float32),
        pltpu.VMEM((MAX_OUT,), jnp.float32),
        pltpu.VMEM((OUTW,), jnp.float32),
        pltpu.VMEM((OUTW,), jnp.float32),
        pltpu.VMEM((OUTW,), jnp.float32),
        pltpu.SemaphoreType.DMA,
        pltpu.SemaphoreType.DMA,
        pltpu.SemaphoreType.DMA,
        pltpu.SemaphoreType.DMA,
    ],
)
def _sc_nms(scores, lvl1, lvl2, boxes, out, *scratch):
    _sc_nms_kernel(scores, lvl1, lvl2, boxes, out, *scratch)


def kernel(classification, regression, anchors):
    scores_t = classification[0].T
    anchors_t = anchors[0].T
    regression_t = regression[0].T
    l1, l2, boxes = _prep(scores_t, anchors_t, regression_t)
    out = _sc_nms(scores_t, l1, l2, boxes)
    outs = out[:, :MAX_OUT]
    final_scores = outs.reshape(-1)
    labels = jnp.broadcast_to(
        jnp.arange(NUM_CLASSES, dtype=jnp.int32)[:, None],
        (NUM_CLASSES, MAX_OUT))
    final_labels = jnp.where(outs > SCORE_THR, labels, -1).reshape(-1)
    final_boxes = (out[:, MAX_OUT:].reshape(NUM_CLASSES, 4, MAX_OUT)
                   .transpose(0, 2, 1).reshape(-1, 4))
    return final_scores, final_labels, final_boxes


# speedup vs baseline: 1.0441x; 1.0441x over previous
import functools

import jax
import jax.numpy as jnp
from jax import lax
from jax.experimental import pallas as pl
from jax.experimental.pallas import tpu as pltpu
from jax.experimental.pallas import tpu_sc as plsc

IMG = 512.0
IOU_THR = 0.5
SCORE_THR = 0.05
MAX_OUT = 64
NUM_CLASSES = 80
N = 20000
NEG = -1e9

L = 16
CH0 = 128
NP = 20480
NL1 = NP // CH0
NL2 = L
OUTW = 5 * MAX_OUT

NUM_SC_CORES = 2
NUM_SUBCORES = 16
NW = NUM_SC_CORES * NUM_SUBCORES
CLASSES_PER_W = (NUM_CLASSES + NW - 1) // NW


def _prep_tc_kernel(s_ref, anc_ref, reg_ref, lvl1_ref, lvl2_ref, boxes_ref):
    s = s_ref[...]
    sp = jnp.concatenate(
        [s, jnp.full((NUM_CLASSES, NP - N), NEG, jnp.float32)], axis=1)
    c1 = jnp.max(sp.reshape(NUM_CLASSES, NL1, CH0), axis=2)
    l1 = jnp.where(c1 > SCORE_THR, c1, NEG)
    lvl1_ref[...] = l1
    c2 = jnp.max(l1.reshape(NUM_CLASSES, 10, L), axis=2)
    lvl2_ref[...] = jnp.concatenate(
        [c2, jnp.full((NUM_CLASSES, NL2 - 10), NEG, jnp.float32)], axis=1)
    a0 = anc_ref[0, :]
    a1 = anc_ref[1, :]
    a2 = anc_ref[2, :]
    a3 = anc_ref[3, :]
    r0 = reg_ref[0, :]
    r1 = reg_ref[1, :]
    r2 = reg_ref[2, :]
    r3 = reg_ref[3, :]
    w = a2 - a0
    h = a3 - a1
    cx = a0 + 0.5 * w
    cy = a1 + 0.5 * h
    dx = r0 * 0.1
    dy = r1 * 0.1
    dw = r2 * 0.2
    dh = r3 * 0.2
    pcx = cx + dx * w
    pcy = cy + dy * h
    pw = jnp.exp(dw) * w
    ph = jnp.exp(dh) * h
    boxes_ref[0, :] = jnp.maximum(pcx - 0.5 * pw, 0.0)
    boxes_ref[1, :] = jnp.maximum(pcy - 0.5 * ph, 0.0)
    boxes_ref[2, :] = jnp.minimum(pcx + 0.5 * pw, IMG)
    boxes_ref[3, :] = jnp.minimum(pcy + 0.5 * ph, IMG)


def _prep(scores_t, anchors_t, regression_t):
    return pl.pallas_call(
        _prep_tc_kernel,
        out_shape=[
            jax.ShapeDtypeStruct((NUM_CLASSES, NL1), jnp.float32),
            jax.ShapeDtypeStruct((NUM_CLASSES, NL2), jnp.float32),
            jax.ShapeDtypeStruct((4, N), jnp.float32),
        ],
    )(scores_t, anchors_t, regression_t)


def _smax(v, perms):
    for p in perms:
        v = jnp.maximum(v, jnp.take(v, p))
    return v


def _sc_nms_kernel(scores_hbm, lvl1_hbm, lvl2_hbm, boxes_hbm, out_hbm,
                   sc_a, sc_b, l1_a, l1_b, l2_a, l2_b,
                   bx1_v, by1_v, bx2_v, by2_v,
                   sxa1, sya1, sxa2, sya2, osa,
                   sxb1, syb1, sxb2, syb2, osb,
                   stg0, stg1, stg2,
                   semb, sem0, sem1, semo):
    wid = lax.axis_index("s") * NUM_SC_CORES + lax.axis_index("c")
    iot = lax.iota(jnp.int32, L)
    perms = [iot ^ (1 << k) for k in range(4)]
    zeros = jnp.zeros((L,), jnp.float32)
    negv = jnp.full((L,), NEG, jnp.float32)
    thrv = jnp.full((L,), SCORE_THR, jnp.float32)
    iouv = jnp.full((L,), IOU_THR, jnp.float32)

    bufs = [(sc_a, l1_a, l2_a, sem0), (sc_b, l1_b, l2_b, sem1)]
    sels = [(sxa1, sya1, sxa2, sya2, osa), (sxb1, syb1, sxb2, syb2, osb)]
    stgs = [stg0, stg1, stg2]

    def fire(c, t):
        sc_v, l1_v, l2_v, sem = bufs[t % 2]
        return [pltpu.async_copy(scores_hbm.at[c], sc_v, sem),
                pltpu.async_copy(lvl1_hbm.at[c], l1_v, sem),
                pltpu.async_copy(lvl2_hbm.at[c], l2_v, sem)]

    def zero_sel(sel):
        for ref in sel:
            for k in range(MAX_OUT // L):
                ref[pl.ds(k * L, L)] = zeros

    def half(count, mv, t, si):
        sc_v, l1_v, l2_v, _ = bufs[t % 2]
        sx1_v, sy1_v, sx2_v, sy2_v, os_v = sels[si]
        v2 = l2_v[...]
        j2 = plsc.all_reduce_ffs(v2 == mv)[0]
        v1 = l1_v[pl.ds(j2 * L, L)]
        j1loc = plsc.all_reduce_ffs(v1 == mv)[0]
        j1 = j2 * L + j1loc
        base0 = j1 * CH0
        r2v = _smax(jnp.where(iot == j2, negv, v2), perms)
        r1v = _smax(jnp.where(iot == j1loc, negv, v1), perms)
        offs = [jnp.minimum(base0 + k * L, N - L) for k in range(CH0 // L)]
        subs = []
        cand = jnp.full((L,), 4096, jnp.int32)
        for k in range(CH0 // L):
            sk = sc_v[pl.ds(offs[k], L)]
            subs.append(sk)
            f = plsc.all_reduce_ffs(sk == mv)
            cand = jnp.minimum(
                cand, jnp.where(f == L, 4096, k * L + f))
        idx_loc = cand[0]
        lane = lax.rem(idx_loc, L)
        k0 = lax.div(idx_loc, L)
        off0 = base0 + k0 * L
        idx = base0 + idx_loc
        sub = sc_v[pl.ds(off0, L)]
        subn = jnp.where(iot == lane, negv, sub)
        sc_v[pl.ds(off0, L)] = subn
        part = jnp.where(jnp.full((L,), offs[0] == off0), subn, subs[0])
        for k in range(1, CH0 // L):
            sk = jnp.where(jnp.full((L,), offs[k] == off0), subn, subs[k])
            part = jnp.maximum(part, sk)
        m1v_raw = _smax(part, perms)
        m1v = jnp.where(m1v_raw > thrv, m1v_raw, negv)
        v1n = jnp.where(iot == j1loc, m1v, v1)
        l1_v[pl.ds(j2 * L, L)] = v1n
        m2v = jnp.maximum(m1v, r1v)
        v2n = jnp.where(iot == j2, m2v, v2)
        l2_v[...] = v2n
        m_next = jnp.maximum(m2v, r2v)
        idxv = jnp.full((L,), idx, jnp.int32)
        cx1 = plsc.load_gather(bx1_v, [idxv])
        cy1 = plsc.load_gather(by1_v, [idxv])
        cx2 = plsc.load_gather(bx2_v, [idxv])
        cy2 = plsc.load_gather(by2_v, [idxv])
        ca = (cx2 - cx1) * (cy2 - cy1)
        acc = zeros
        for k in range(MAX_OUT // L):
            sx1 = sx1_v[pl.ds(k * L, L)]
            sy1 = sy1_v[pl.ds(k * L, L)]
            sx2 = sx2_v[pl.ds(k * L, L)]
            sy2 = sy2_v[pl.ds(k * L, L)]
            xx1 = jnp.maximum(sx1, cx1)
            yy1 = jnp.maximum(sy1, cy1)
            xx2 = jnp.minimum(sx2, cx2)
            yy2 = jnp.minimum(sy2, cy2)
            inter = jnp.maximum(xx2 - xx1, 0.0) * jnp.maximum(yy2 - yy1, 0.0)
            sa = (sx2 - sx1) * (sy2 - sy1)
            iou = inter / (ca + sa - inter + 1e-8)
            acc = jnp.maximum(acc, iou)
        ok = jnp.logical_not(jnp.any(acc > iouv))
        coff = lax.div(count, L) * L
        lmask = jnp.logical_and(iot == (count - coff),
                                jnp.full((L,), ok))
        for ref, val in ((sx1_v, cx1), (sy1_v, cy1),
                         (sx2_v, cx2), (sy2_v, cy2), (os_v, mv)):
            w = ref[pl.ds(coff, L)]
            ref[pl.ds(coff, L)] = jnp.where(lmask, val, w)
        return count + ok.astype(jnp.int32), m_next

    def stage_out(c, t, si):
        stg = stgs[t]
        sx1_v, sy1_v, sx2_v, sy2_v, os_v = sels[si]
        for i, ref in enumerate((os_v, sx1_v, sy1_v, sx2_v, sy2_v)):
            for k in range(MAX_OUT // L):
                stg[pl.ds(i * MAX_OUT + k * L, L)] = ref[pl.ds(k * L, L)]
        return pltpu.async_copy(stg, out_hbm.at[c], semo)

    def walk(c, t):
        _, l1_v, l2_v, _ = bufs[t % 2]
        zero_sel(sels[0])
        m0 = _smax(l2_v[...], perms)

        def cond(carry):
            count, mv = carry
            return jnp.logical_and(count < MAX_OUT, jnp.any(mv > thrv))

        def body(carry):
            return half(carry[0], carry[1], t, 0)

        lax.while_loop(cond, body, (jnp.int32(0), m0))
        return stage_out(c, t, 0)

    hb = [pltpu.async_copy(boxes_hbm.at[0], bx1_v, semb),
          pltpu.async_copy(boxes_hbm.at[1], by1_v, semb),
          pltpu.async_copy(boxes_hbm.at[2], bx2_v, semb),
          pltpu.async_copy(boxes_hbm.at[3], by2_v, semb)]
    h0 = fire(wid, 0)
    h1 = fire(wid + NW, 1)
    for h in hb + h0:
        h.wait()

    ho0 = walk(wid, 0)

    has3 = wid + 2 * NW < NUM_CLASSES
    c2 = jnp.minimum(wid + 2 * NW, NUM_CLASSES - 1)
    h2 = fire(c2, 2)
    for h in h1:
        h.wait()
    ho1 = walk(wid + NW, 1)
    for h in h2:
        h.wait()

    @pl.when(has3)
    def _():
        walk(wid + 2 * NW, 2).wait()
    ho0.wait()
    ho1.wait()


@functools.partial(
    pl.kernel,
    out_type=jax.ShapeDtypeStruct((NUM_CLASSES, OUTW), jnp.float32),
    mesh=plsc.VectorSubcoreMesh(core_axis_name="c", subcore_axis_name="s",
                                num_cores=NUM_SC_CORES,
                                num_subcores=NUM_SUBCORES),
    compiler_params=pltpu.CompilerParams(needs_layout_passes=False),
    scratch_types=[
        pltpu.VMEM((N,), jnp.float32),
        pltpu.VMEM((N,), jnp.float32),
        pltpu.VMEM((NL1,), jnp.float32),
        pltpu.VMEM((NL1,), jnp.float32),
        pltpu.VMEM((NL2,), jnp.float32),
        pltpu.VMEM((NL2,), jnp.float32),
        pltpu.VMEM((N,), jnp.float32),
        pltpu.VMEM((N,), jnp.float32),
        pltpu.VMEM((N,), jnp.float32),
        pltpu.VMEM((N,), jnp.float32),
        pltpu.VMEM((MAX_OUT,), jnp.float32),
        pltpu.VMEM((MAX_OUT,), jnp.float32),
        pltpu.VMEM((MAX_OUT,), jnp.float32),
        pltpu.VMEM((MAX_OUT,), jnp.float32),
        pltpu.VMEM((MAX_OUT,), jnp.float32),
        pltpu.VMEM((MAX_OUT,), jnp.float32),
        pltpu.VMEM((MAX_OUT,), jnp.float32),
        pltpu.VMEM((MAX_OUT,), jnp.float32),
        pltpu.VMEM((MAX_OUT,), jnp.float32),
        pltpu.VMEM((MAX_OUT,), jnp.float32),
        pltpu.VMEM((OUTW,), jnp.float32),
        pltpu.VMEM((OUTW,), jnp.float32),
        pltpu.VMEM((OUTW,), jnp.float32),
        pltpu.SemaphoreType.DMA,
        pltpu.SemaphoreType.DMA,
        pltpu.SemaphoreType.DMA,
        pltpu.SemaphoreType.DMA,
    ],
)
def _sc_nms(scores, lvl1, lvl2, boxes, out, *scratch):
    _sc_nms_kernel(scores, lvl1, lvl2, boxes, out, *scratch)


def kernel(classification, regression, anchors):
    scores_t = classification[0].T
    anchors_t = anchors[0].T
    regression_t = regression[0].T
    l1, l2, boxes = _prep(scores_t, anchors_t, regression_t)
    out = _sc_nms(scores_t, l1, l2, boxes)
    outs = out[:, :MAX_OUT]
    final_scores = outs.reshape(-1)
    labels = jnp.broadcast_to(
        jnp.arange(NUM_CLASSES, dtype=jnp.int32)[:, None],
        (NUM_CLASSES, MAX_OUT))
    final_labels = jnp.where(outs > SCORE_THR, labels, -1).reshape(-1)
    final_boxes = (out[:, MAX_OUT:].reshape(NUM_CLASSES, 4, MAX_OUT)
                   .transpose(0, 2, 1).reshape(-1, 4))
    return final_scores, final_labels, final_boxes


# speedup vs baseline: 1.0750x; 1.0296x over previous
import functools

import jax
import jax.numpy as jnp
from jax import lax
from jax.experimental import pallas as pl
from jax.experimental.pallas import tpu as pltpu
from jax.experimental.pallas import tpu_sc as plsc

IMG = 512.0
IOU_THR = 0.5
SCORE_THR = 0.05
MAX_OUT = 64
NUM_CLASSES = 80
N = 20000
NEG = -1e9

L = 16
CH0 = 128
NP = 20480
NL1 = NP // CH0
NL2 = L
OUTW = 5 * MAX_OUT

NUM_SC_CORES = 2
NUM_SUBCORES = 16
NW = NUM_SC_CORES * NUM_SUBCORES
CLASSES_PER_W = (NUM_CLASSES + NW - 1) // NW


def _prep_tc_kernel(s_ref, anc_ref, reg_ref, lvl1_ref, lvl2_ref, boxes_ref):
    s = s_ref[...]
    nfull = N // CH0
    c1a = jnp.max(s[:, :nfull * CH0].reshape(NUM_CLASSES, nfull, CH0),
                  axis=2)
    c1b = jnp.max(s[:, nfull * CH0:], axis=1, keepdims=True)
    c1 = jnp.concatenate(
        [c1a, c1b,
         jnp.full((NUM_CLASSES, NL1 - nfull - 1), NEG, jnp.float32)], axis=1)
    l1 = jnp.where(c1 > SCORE_THR, c1, NEG)
    lvl1_ref[...] = l1
    c2 = jnp.max(l1.reshape(NUM_CLASSES, 10, L), axis=2)
    lvl2_ref[...] = jnp.concatenate(
        [c2, jnp.full((NUM_CLASSES, NL2 - 10), NEG, jnp.float32)], axis=1)
    a0 = anc_ref[0, :]
    a1 = anc_ref[1, :]
    a2 = anc_ref[2, :]
    a3 = anc_ref[3, :]
    r0 = reg_ref[0, :]
    r1 = reg_ref[1, :]
    r2 = reg_ref[2, :]
    r3 = reg_ref[3, :]
    w = a2 - a0
    h = a3 - a1
    cx = a0 + 0.5 * w
    cy = a1 + 0.5 * h
    dx = r0 * 0.1
    dy = r1 * 0.1
    dw = r2 * 0.2
    dh = r3 * 0.2
    pcx = cx + dx * w
    pcy = cy + dy * h
    pw = jnp.exp(dw) * w
    ph = jnp.exp(dh) * h
    boxes_ref[0, :] = jnp.maximum(pcx - 0.5 * pw, 0.0)
    boxes_ref[1, :] = jnp.maximum(pcy - 0.5 * ph, 0.0)
    boxes_ref[2, :] = jnp.minimum(pcx + 0.5 * pw, IMG)
    boxes_ref[3, :] = jnp.minimum(pcy + 0.5 * ph, IMG)


def _prep(scores_t, anchors_t, regression_t):
    return pl.pallas_call(
        _prep_tc_kernel,
        out_shape=[
            jax.ShapeDtypeStruct((NUM_CLASSES, NL1), jnp.float32),
            jax.ShapeDtypeStruct((NUM_CLASSES, NL2), jnp.float32),
            jax.ShapeDtypeStruct((4, N), jnp.float32),
        ],
    )(scores_t, anchors_t, regression_t)


def _smax(v, perms):
    for p in perms:
        v = jnp.maximum(v, jnp.take(v, p))
    return v


def _sc_nms_kernel(scores_hbm, lvl1_hbm, lvl2_hbm, boxes_hbm, out_hbm,
                   sc_a, sc_b, l1_a, l1_b, l2_a, l2_b,
                   bx1_v, by1_v, bx2_v, by2_v,
                   sxa1, sya1, sxa2, sya2, osa,
                   sxb1, syb1, sxb2, syb2, osb,
                   stg0, stg1, stg2,
                   semb, sem0, sem1, semo):
    wid = lax.axis_index("s") * NUM_SC_CORES + lax.axis_index("c")
    iot = lax.iota(jnp.int32, L)
    perms = [iot ^ (1 << k) for k in range(4)]
    zeros = jnp.zeros((L,), jnp.float32)
    negv = jnp.full((L,), NEG, jnp.float32)
    thrv = jnp.full((L,), SCORE_THR, jnp.float32)
    iouv = jnp.full((L,), IOU_THR, jnp.float32)

    bufs = [(sc_a, l1_a, l2_a, sem0), (sc_b, l1_b, l2_b, sem1)]
    sels = [(sxa1, sya1, sxa2, sya2, osa), (sxb1, syb1, sxb2, syb2, osb)]
    stgs = [stg0, stg1, stg2]

    def fire(c, t):
        sc_v, l1_v, l2_v, sem = bufs[t % 2]
        return [pltpu.async_copy(scores_hbm.at[c], sc_v, sem),
                pltpu.async_copy(lvl1_hbm.at[c], l1_v, sem),
                pltpu.async_copy(lvl2_hbm.at[c], l2_v, sem)]

    def zero_sel(sel):
        for ref in sel:
            for k in range(MAX_OUT // L):
                ref[pl.ds(k * L, L)] = zeros

    def half(count, mv, t, si):
        sc_v, l1_v, l2_v, _ = bufs[t % 2]
        sx1_v, sy1_v, sx2_v, sy2_v, os_v = sels[si]
        v2 = l2_v[...]
        j2 = plsc.all_reduce_ffs(v2 == mv)[0]
        v1 = l1_v[pl.ds(j2 * L, L)]
        j1loc = plsc.all_reduce_ffs(v1 == mv)[0]
        j1 = j2 * L + j1loc
        base0 = j1 * CH0
        r2v = _smax(jnp.where(iot == j2, negv, v2), perms)
        r1v = _smax(jnp.where(iot == j1loc, negv, v1), perms)
        offs = [jnp.minimum(base0 + k * L, N - L) for k in range(CH0 // L)]
        subs = []
        cand = jnp.full((L,), 4096, jnp.int32)
        for k in range(CH0 // L):
            sk = sc_v[pl.ds(offs[k], L)]
            subs.append(sk)
            f = plsc.all_reduce_ffs(sk == mv)
            cand = jnp.minimum(
                cand, jnp.where(f == L, 4096, k * L + f))
        idx_loc = cand[0]
        lane = lax.rem(idx_loc, L)
        k0 = lax.div(idx_loc, L)
        off0 = base0 + k0 * L
        idx = base0 + idx_loc
        sub = sc_v[pl.ds(off0, L)]
        subn = jnp.where(iot == lane, negv, sub)
        sc_v[pl.ds(off0, L)] = subn
        part = jnp.where(jnp.full((L,), offs[0] == off0), subn, subs[0])
        for k in range(1, CH0 // L):
            sk = jnp.where(jnp.full((L,), offs[k] == off0), subn, subs[k])
            part = jnp.maximum(part, sk)
        m1v_raw = _smax(part, perms)
        m1v = jnp.where(m1v_raw > thrv, m1v_raw, negv)
        v1n = jnp.where(iot == j1loc, m1v, v1)
        l1_v[pl.ds(j2 * L, L)] = v1n
        m2v = jnp.maximum(m1v, r1v)
        v2n = jnp.where(iot == j2, m2v, v2)
        l2_v[...] = v2n
        m_next = jnp.maximum(m2v, r2v)
        idxv = jnp.full((L,), idx, jnp.int32)
        cx1 = plsc.load_gather(bx1_v, [idxv])
        cy1 = plsc.load_gather(by1_v, [idxv])
        cx2 = plsc.load_gather(bx2_v, [idxv])
        cy2 = plsc.load_gather(by2_v, [idxv])
        ca = (cx2 - cx1) * (cy2 - cy1)
        acc = zeros
        for k in range(MAX_OUT // L):
            sx1 = sx1_v[pl.ds(k * L, L)]
            sy1 = sy1_v[pl.ds(k * L, L)]
            sx2 = sx2_v[pl.ds(k * L, L)]
            sy2 = sy2_v[pl.ds(k * L, L)]
            xx1 = jnp.maximum(sx1, cx1)
            yy1 = jnp.maximum(sy1, cy1)
            xx2 = jnp.minimum(sx2, cx2)
            yy2 = jnp.minimum(sy2, cy2)
            inter = jnp.maximum(xx2 - xx1, 0.0) * jnp.maximum(yy2 - yy1, 0.0)
            sa = (sx2 - sx1) * (sy2 - sy1)
            iou = inter / (ca + sa - inter + 1e-8)
            acc = jnp.maximum(acc, iou)
        ok = jnp.logical_not(jnp.any(acc > iouv))
        coff = lax.div(count, L) * L
        lmask = jnp.logical_and(iot == (count - coff),
                                jnp.full((L,), ok))
        for ref, val in ((sx1_v, cx1), (sy1_v, cy1),
                         (sx2_v, cx2), (sy2_v, cy2), (os_v, mv)):
            w = ref[pl.ds(coff, L)]
            ref[pl.ds(coff, L)] = jnp.where(lmask, val, w)
        return count + ok.astype(jnp.int32), m_next

    def stage_out(c, t, si):
        stg = stgs[t]
        sx1_v, sy1_v, sx2_v, sy2_v, os_v = sels[si]
        for i, ref in enumerate((os_v, sx1_v, sy1_v, sx2_v, sy2_v)):
            for k in range(MAX_OUT // L):
                stg[pl.ds(i * MAX_OUT + k * L, L)] = ref[pl.ds(k * L, L)]
        return pltpu.async_copy(stg, out_hbm.at[c], semo)

    def walk(c, t):
        _, l1_v, l2_v, _ = bufs[t % 2]
        zero_sel(sels[0])
        m0 = _smax(l2_v[...], perms)

        def cond(carry):
            count, mv = carry
            return jnp.logical_and(count < MAX_OUT, jnp.any(mv > thrv))

        def body(carry):
            return half(carry[0], carry[1], t, 0)

        lax.while_loop(cond, body, (jnp.int32(0), m0))
        return stage_out(c, t, 0)

    hb = [pltpu.async_copy(boxes_hbm.at[0], bx1_v, semb),
          pltpu.async_copy(boxes_hbm.at[1], by1_v, semb),
          pltpu.async_copy(boxes_hbm.at[2], bx2_v, semb),
          pltpu.async_copy(boxes_hbm.at[3], by2_v, semb)]
    h0 = fire(wid, 0)
    h1 = fire(wid + NW, 1)
    for h in hb + h0:
        h.wait()

    ho0 = walk(wid, 0)

    has3 = wid + 2 * NW < NUM_CLASSES
    c2 = jnp.minimum(wid + 2 * NW, NUM_CLASSES - 1)
    h2 = fire(c2, 2)
    for h in h1:
        h.wait()
    ho1 = walk(wid + NW, 1)
    for h in h2:
        h.wait()

    @pl.when(has3)
    def _():
        walk(wid + 2 * NW, 2).wait()
    ho0.wait()
    ho1.wait()


@functools.partial(
    pl.kernel,
    out_type=jax.ShapeDtypeStruct((NUM_CLASSES, OUTW), jnp.float32),
    mesh=plsc.VectorSubcoreMesh(core_axis_name="c", subcore_axis_name="s",
                                num_cores=NUM_SC_CORES,
                                num_subcores=NUM_SUBCORES),
    compiler_params=pltpu.CompilerParams(needs_layout_passes=False),
    scratch_types=[
        pltpu.VMEM((N,), jnp.float32),
        pltpu.VMEM((N,), jnp.float32),
        pltpu.VMEM((NL1,), jnp.float32),
        pltpu.VMEM((NL1,), jnp.float32),
        pltpu.VMEM((NL2,), jnp.float32),
        pltpu.VMEM((NL2,), jnp.float32),
        pltpu.VMEM((N,), jnp.float32),
        pltpu.VMEM((N,), jnp.float32),
        pltpu.VMEM((N,), jnp.float32),
        pltpu.VMEM((N,), jnp.float32),
        pltpu.VMEM((MAX_OUT,), jnp.float32),
        pltpu.VMEM((MAX_OUT,), jnp.float32),
        pltpu.VMEM((MAX_OUT,), jnp.float32),
        pltpu.VMEM((MAX_OUT,), jnp.float32),
        pltpu.VMEM((MAX_OUT,), jnp.float32),
        pltpu.VMEM((MAX_OUT,), jnp.float32),
        pltpu.VMEM((MAX_OUT,), jnp.float32),
        pltpu.VMEM((MAX_OUT,), jnp.float32),
        pltpu.VMEM((MAX_OUT,), jnp.float32),
        pltpu.VMEM((MAX_OUT,), jnp.float32),
        pltpu.VMEM((OUTW,), jnp.float32),
        pltpu.VMEM((OUTW,), jnp.float32),
        pltpu.VMEM((OUTW,), jnp.float32),
        pltpu.SemaphoreType.DMA,
        pltpu.SemaphoreType.DMA,
        pltpu.SemaphoreType.DMA,
        pltpu.SemaphoreType.DMA,
    ],
)
def _sc_nms(scores, lvl1, lvl2, boxes, out, *scratch):
    _sc_nms_kernel(scores, lvl1, lvl2, boxes, out, *scratch)


def kernel(classification, regression, anchors):
    scores_t = classification[0].T
    anchors_t = anchors[0].T
    regression_t = regression[0].T
    l1, l2, boxes = _prep(scores_t, anchors_t, regression_t)
    out = _sc_nms(scores_t, l1, l2, boxes)
    outs = out[:, :MAX_OUT]
    final_scores = outs.reshape(-1)
    labels = jnp.broadcast_to(
        jnp.arange(NUM_CLASSES, dtype=jnp.int32)[:, None],
        (NUM_CLASSES, MAX_OUT))
    final_labels = jnp.where(outs > SCORE_THR, labels, -1).reshape(-1)
    final_boxes = (out[:, MAX_OUT:].reshape(NUM_CLASSES, 4, MAX_OUT)
                   .transpose(0, 2, 1).reshape(-1, 4))
    return final_scores, final_labels, final_boxes
